# Initial kernel scaffold; baseline (speedup 1.0000x reference)
import functools

import jax
import jax.numpy as jnp
from jax import lax
from jax.experimental import pallas as pl
from jax.experimental.pallas import tpu as pltpu
from jax.experimental.pallas import tpu_sc as plsc

BATCH = 16384
FIELDS = 26
EMBED_DIM = 32
WEIGHT = 0.5
LANES = 16
NUM_CORES = 2
NUM_SUBCORES = 16
NW = NUM_CORES * NUM_SUBCORES
ROWS_PER_W = BATCH // NW
CHUNK_ROWS = 64
NCHUNKS = ROWS_PER_W // CHUNK_ROWS
IDX_PER_CHUNK = CHUNK_ROWS * FIELDS
IDX_TILE = 128
NIDX_TILES = IDX_PER_CHUNK // IDX_TILE


def _fm_body(x_hbm, w0_hbm, w1_hbm, v_hbm, out_hbm,
             idx_v, rows_v, w1_v, out_v, w0_v, gsem, wsem):
    wid = lax.axis_index("s") * NUM_CORES + lax.axis_index("c")
    pltpu.sync_copy(w0_hbm, w0_v)
    w0vec = w0_v[...]
    w1_v[pl.ds(IDX_PER_CHUNK, LANES)] = jnp.zeros((LANES,), jnp.float32)
    iota = lax.iota(jnp.int32, (LANES,))
    mask_tail = (iota < (FIELDS - LANES)).astype(jnp.float32)

    def do_chunk(c, carry):
        row_base = wid * ROWS_PER_W + c * CHUNK_ROWS
        xrow = wid * (ROWS_PER_W * FIELDS // IDX_TILE) + c * NIDX_TILES
        pltpu.sync_copy(x_hbm.at[pl.ds(xrow, NIDX_TILES)], idx_v)
        copies = []
        for j in range(NIDX_TILES):
            copies.append(pltpu.async_copy(
                v_hbm.at[idx_v.at[j]],
                rows_v.at[pl.ds(j * IDX_TILE, IDX_TILE)], gsem))
            copies.append(pltpu.async_copy(
                w1_hbm.at[idx_v.at[j]],
                w1_v.at[pl.ds(j * IDX_TILE, IDX_TILE)], wsem))
        for cp in copies:
            cp.wait()

        def row_body(b, carry2):
            rbase = b * FIELDS
            acc0 = jnp.zeros((LANES,), jnp.float32)
            acc1 = jnp.zeros((LANES,), jnp.float32)
            sq0 = jnp.zeros((LANES,), jnp.float32)
            sq1 = jnp.zeros((LANES,), jnp.float32)
            for f in range(FIELDS):
                v0 = rows_v[rbase + f, pl.ds(0, LANES)]
                v1 = rows_v[rbase + f, pl.ds(LANES, LANES)]
                acc0 = acc0 + v0
                acc1 = acc1 + v1
                sq0 = sq0 + v0 * v0
                sq1 = sq1 + v1 * v1
            l0 = plsc.load_gather(w1_v, [rbase + iota])
            l1 = plsc.load_gather(w1_v, [rbase + LANES + iota]) * mask_tail
            lin = jnp.sum(l0 + l1)
            linv = jnp.full((LANES,), lin, jnp.float32) + w0vec
            out_v[b, pl.ds(0, LANES)] = linv + WEIGHT * (acc0 * acc0 + sq0)
            out_v[b, pl.ds(LANES, LANES)] = linv + WEIGHT * (acc1 * acc1 + sq1)
            return carry2

        lax.fori_loop(0, CHUNK_ROWS, row_body, 0)
        pltpu.sync_copy(out_v, out_hbm.at[pl.ds(row_base, CHUNK_ROWS)])
        return carry

    lax.fori_loop(0, NCHUNKS, do_chunk, 0)


@jax.jit
def _fm(x2, w0b, w1f, V):
    mesh = plsc.VectorSubcoreMesh(core_axis_name="c", subcore_axis_name="s")
    f = functools.partial(
        pl.kernel,
        out_type=jax.ShapeDtypeStruct((BATCH, EMBED_DIM), jnp.float32),
        mesh=mesh,
        scratch_types=[
            pltpu.VMEM((NIDX_TILES, IDX_TILE), jnp.int32),
            pltpu.VMEM((IDX_PER_CHUNK, EMBED_DIM), jnp.float32),
            pltpu.VMEM((IDX_PER_CHUNK + LANES,), jnp.float32),
            pltpu.VMEM((CHUNK_ROWS, EMBED_DIM), jnp.float32),
            pltpu.VMEM((LANES,), jnp.float32),
            pltpu.SemaphoreType.DMA,
            pltpu.SemaphoreType.DMA,
        ],
    )(_fm_body)
    return f(x2, w0b, w1f, V)


def kernel(x, W0, W1, V):
    x2 = x.reshape(BATCH * FIELDS // IDX_TILE, IDX_TILE).astype(jnp.int32)
    w0b = jnp.broadcast_to(W0.astype(jnp.float32), (LANES,))
    w1f = W1.reshape(-1)
    return _fm(x2, w0b, w1f, V)


# speedup vs baseline: 2.1353x; 2.1353x over previous
import functools

import jax
import jax.numpy as jnp
from jax import lax
from jax.experimental import pallas as pl
from jax.experimental.pallas import tpu as pltpu
from jax.experimental.pallas import tpu_sc as plsc

BATCH = 16384
FIELDS = 26
EMBED_DIM = 32
WEIGHT = 0.5
LANES = 16
NUM_CORES = 2
NUM_SUBCORES = 16
NW = NUM_CORES * NUM_SUBCORES
ROWS_PER_W = BATCH // NW
CHUNK_ROWS = 64
NCHUNKS = ROWS_PER_W // CHUNK_ROWS
IDX_PER_CHUNK = CHUNK_ROWS * FIELDS
IDX_TILE = 128
NIDX_TILES = IDX_PER_CHUNK // IDX_TILE


def _fm_body(x_hbm, w0_hbm, w1_hbm, v_hbm, out_hbm,
             idx_v, rows_v, w1_v, out_v, w0_v, gsem, wsem):
    wid = lax.axis_index("s") * NUM_CORES + lax.axis_index("c")
    pltpu.sync_copy(w0_hbm, w0_v)
    w0vec = w0_v[...]
    w1_v[pl.ds(IDX_PER_CHUNK, LANES)] = jnp.zeros((LANES,), jnp.float32)
    iota = lax.iota(jnp.int32, LANES)
    mask_tail = (iota < (FIELDS - LANES)).astype(jnp.float32)

    def do_chunk(c, carry):
        row_base = wid * ROWS_PER_W + c * CHUNK_ROWS
        xoff = (wid * ROWS_PER_W + c * CHUNK_ROWS) * FIELDS
        pltpu.sync_copy(x_hbm.at[pl.ds(xoff, IDX_PER_CHUNK)], idx_v)
        copies = []
        for j in range(NIDX_TILES):
            copies.append(pltpu.async_copy(
                v_hbm.at[idx_v.at[pl.ds(j * IDX_TILE, IDX_TILE)]],
                rows_v.at[pl.ds(j * IDX_TILE, IDX_TILE)], gsem))
            copies.append(pltpu.async_copy(
                w1_hbm.at[idx_v.at[pl.ds(j * IDX_TILE, IDX_TILE)]],
                w1_v.at[pl.ds(j * IDX_TILE, IDX_TILE)], wsem))
        for cp in copies:
            cp.wait()

        def row_body(b, carry2):
            rbase = b * FIELDS
            acc0 = jnp.zeros((LANES,), jnp.float32)
            acc1 = jnp.zeros((LANES,), jnp.float32)
            sq0 = jnp.zeros((LANES,), jnp.float32)
            sq1 = jnp.zeros((LANES,), jnp.float32)
            for f in range(FIELDS):
                v0 = rows_v[rbase + f, pl.ds(0, LANES)]
                v1 = rows_v[rbase + f, pl.ds(LANES, LANES)]
                acc0 = acc0 + v0
                acc1 = acc1 + v1
                sq0 = sq0 + v0 * v0
                sq1 = sq1 + v1 * v1
            l0 = plsc.load_gather(w1_v, [rbase + iota])
            l1 = plsc.load_gather(w1_v, [rbase + LANES + iota]) * mask_tail
            lin = jnp.sum(l0 + l1)
            linv = jnp.full((LANES,), lin, jnp.float32) + w0vec
            out_v[b, pl.ds(0, LANES)] = linv + WEIGHT * (acc0 * acc0 + sq0)
            out_v[b, pl.ds(LANES, LANES)] = linv + WEIGHT * (acc1 * acc1 + sq1)
            return carry2

        lax.fori_loop(0, CHUNK_ROWS, row_body, 0)
        pltpu.sync_copy(out_v, out_hbm.at[pl.ds(row_base, CHUNK_ROWS)])
        return carry

    lax.fori_loop(0, NCHUNKS, do_chunk, 0)


@jax.jit
def _fm(x2, w0b, w1f, V):
    mesh = plsc.VectorSubcoreMesh(core_axis_name="c", subcore_axis_name="s")
    f = functools.partial(
        pl.kernel,
        out_type=jax.ShapeDtypeStruct((BATCH, EMBED_DIM), jnp.float32),
        mesh=mesh,
        compiler_params=pltpu.CompilerParams(
            use_tc_tiling_on_sc=False, needs_layout_passes=False),
        scratch_types=[
            pltpu.VMEM((IDX_PER_CHUNK,), jnp.int32),
            pltpu.VMEM((IDX_PER_CHUNK, EMBED_DIM), jnp.float32),
            pltpu.VMEM((IDX_PER_CHUNK + LANES,), jnp.float32),
            pltpu.VMEM((CHUNK_ROWS, EMBED_DIM), jnp.float32),
            pltpu.VMEM((LANES,), jnp.float32),
            pltpu.SemaphoreType.DMA,
            pltpu.SemaphoreType.DMA,
        ],
    )(_fm_body)
    return f(x2, w0b, w1f, V)


def kernel(x, W0, W1, V):
    x2 = x.reshape(BATCH * FIELDS).astype(jnp.int32)
    w0b = jnp.broadcast_to(W0.astype(jnp.float32), (LANES,))
    w1f = W1.reshape(-1)
    return _fm(x2, w0b, w1f, V)
